# baseline (speedup 1.0000x reference)
import jax
import jax.numpy as jnp
from jax import lax
from jax.experimental import pallas as pl
from jax.experimental.pallas import tpu as pltpu
from jax.experimental.pallas import tpu_sc as plsc

_SMOOTHING = 0.1
_L = 16


def kernel(target, pred):
    b, k = pred.shape
    low = _SMOOTHING / k
    hi = low + (1.0 - _SMOOTHING)

    mesh = plsc.VectorSubcoreMesh(core_axis_name="c", subcore_axis_name="s")
    nw = mesh.num_cores * mesh.num_subcores
    rpw = b // nw

    def body(target_hbm, out_hbm, buf, tgt_v, sem):
        wid = lax.axis_index("s") * mesh.num_cores + lax.axis_index("c")
        base = wid * rpw
        pltpu.sync_copy(target_hbm.at[pl.ds(base, rpw)], tgt_v)

        lane_ids = jnp.arange(_L, dtype=jnp.int32)
        low_v = jnp.full((_L,), low, jnp.float32)
        hi_v = jnp.full((_L,), hi, jnp.float32)

        def fill(i, carry):
            buf[pl.ds(i * _L, _L)] = low_v
            return carry

        lax.fori_loop(0, k // _L, fill, 0)

        def per_row(i, carry):
            tv = tgt_v[pl.ds((i // _L) * _L, _L)]
            mask = lane_ids == (i % _L)
            plsc.store_scatter(buf, [tv], hi_v, mask=mask)
            cp = pltpu.make_async_copy(buf, out_hbm.at[base + i], sem)
            cp.start()
            cp.wait()
            plsc.store_scatter(buf, [tv], low_v, mask=mask)
            return carry

        lax.fori_loop(0, rpw, per_row, 0)

    f = pl.kernel(
        body,
        out_type=jax.ShapeDtypeStruct((b, k), jnp.float32),
        mesh=mesh,
        scratch_types=[
            pltpu.VMEM((k,), jnp.float32),
            pltpu.VMEM((rpw,), jnp.int32),
            pltpu.SemaphoreType.DMA,
        ],
        compiler_params=pltpu.CompilerParams(needs_layout_passes=False),
    )
    return f(target)
